# Initial kernel scaffold; baseline (speedup 1.0000x reference)
import jax
import jax.numpy as jnp
from jax.experimental import pallas as pl


def kernel(visual_embed, textual_embed, labels, W):
    raise NotImplementedError("write your pallas kernel here")


# speedup vs baseline: 2.2016x; 2.2016x over previous
import functools

import jax
import jax.numpy as jnp
from jax.experimental import pallas as pl
from jax.experimental.pallas import tpu as pltpu

SCALE = 28.0
ALPHA = 0.6
BETA = 0.4
SCALE_POS = 10.0
SCALE_NEG = 40.0


def _loss_kernel(num_classes, grid_n, tile_c,
                 x_ref, w_ref, lab_ref, labrow_ref, tt_ref,
                 out_ref, acc_se_ref, acc_ll_ref):
    i = pl.program_id(0)

    @pl.when(i == 0)
    def _init():
        acc_se_ref[...] = jnp.zeros_like(acc_se_ref)
        acc_ll_ref[...] = jnp.zeros_like(acc_ll_ref)

    w = w_ref[...]
    sumsq = jnp.sum(w * w, axis=0, keepdims=True)
    scale = SCALE * jax.lax.rsqrt(jnp.maximum(sumsq, 1e-30))

    x = x_ref[...]
    dots = jax.lax.dot_general(
        x, w.astype(jnp.bfloat16),
        (((1,), (0,)), ((), ())),
        preferred_element_type=jnp.float32)
    logits = dots * scale

    cid = jax.lax.broadcasted_iota(jnp.int32, (1, tile_c), 1) + i * tile_c
    valid = cid < num_classes
    e = jnp.where(valid, jnp.exp(logits), 0.0)
    acc_se_ref[...] += jnp.sum(e, axis=1, keepdims=True)

    lmask = lab_ref[...] == cid
    acc_ll_ref[...] += jnp.sum(jnp.where(lmask, logits, 0.0),
                               axis=1, keepdims=True)

    @pl.when(i == grid_n - 1)
    def _finalize():
        b = x_ref.shape[0] // 2
        ce = jnp.log(acc_se_ref[...]) - acc_ll_ref[...]
        v_loss = jnp.sum(ce[:b]) / b
        t_loss = jnp.sum(ce[b:]) / b

        sim = jax.lax.dot_general(
            x_ref[0:b, :], tt_ref[...],
            (((1,), (0,)), ((), ())),
            preferred_element_type=jnp.float32)
        lmat = lab_ref[0:b, :] == labrow_ref[0:1, :]
        loss_pos = jnp.log1p(jnp.exp(-SCALE_POS * (sim - ALPHA)))
        loss_neg = jnp.log1p(jnp.exp(SCALE_NEG * (sim - BETA)))
        ga = 2.0 * jnp.sum(jnp.where(lmat, loss_pos, loss_neg)) / b

        col = jax.lax.broadcasted_iota(jnp.int32, (8, 128), 1)
        res = jnp.where(col == 0, v_loss + t_loss,
              jnp.where(col == 1, ga,
              jnp.where(col == 2, v_loss, t_loss)))
        out_ref[...] = res


def kernel(visual_embed, textual_embed, labels, W):
    batch, feat = visual_embed.shape
    num_classes = W.shape[1]
    tile_c = 1024
    grid_n = (num_classes + tile_c - 1) // tile_c

    x = jnp.concatenate([visual_embed, textual_embed], axis=0)
    x = x.astype(jnp.bfloat16)
    tt = textual_embed.T.astype(jnp.bfloat16)
    lab = jnp.concatenate([labels, labels]).astype(jnp.int32)
    lab = lab.reshape(2 * batch, 1)
    labrow = jnp.broadcast_to(labels.astype(jnp.int32)[None, :], (8, batch))

    out = pl.pallas_call(
        functools.partial(_loss_kernel, num_classes, grid_n, tile_c),
        grid=(grid_n,),
        in_specs=[
            pl.BlockSpec((2 * batch, feat), lambda i: (0, 0)),
            pl.BlockSpec((feat, tile_c), lambda i: (0, i)),
            pl.BlockSpec((2 * batch, 1), lambda i: (0, 0)),
            pl.BlockSpec((8, batch), lambda i: (0, 0)),
            pl.BlockSpec((feat, batch), lambda i: (0, 0)),
        ],
        out_specs=pl.BlockSpec((8, 128), lambda i: (0, 0)),
        out_shape=jax.ShapeDtypeStruct((8, 128), jnp.float32),
        scratch_shapes=[
            pltpu.VMEM((2 * batch, 1), jnp.float32),
            pltpu.VMEM((2 * batch, 1), jnp.float32),
        ],
    )(x, W, lab, labrow, tt)

    instance_loss = out[0, 0]
    global_align_loss = out[0, 1]
    v_loss = out[0, 2]
    t_loss = out[0, 3]
    return (instance_loss, global_align_loss, v_loss, t_loss)


# speedup vs baseline: 2.3131x; 1.0506x over previous
import functools

import jax
import jax.numpy as jnp
from jax.experimental import pallas as pl
from jax.experimental.pallas import tpu as pltpu

SCALE = 28.0
ALPHA = 0.6
BETA = 0.4
SCALE_POS = 10.0
SCALE_NEG = 40.0


def _loss_kernel(num_classes, grid_n, tile_c,
                 x_ref, w_ref, lab_ref, labrow_ref, tt_ref,
                 out_ref, acc_se_ref, acc_ll_ref):
    i = pl.program_id(0)

    @pl.when(i == 0)
    def _init():
        acc_se_ref[...] = jnp.zeros_like(acc_se_ref)
        acc_ll_ref[...] = jnp.zeros_like(acc_ll_ref)

    cid = jax.lax.broadcasted_iota(jnp.int32, (1, tile_c), 1) + i * tile_c
    valid = cid < num_classes

    w = jnp.where(valid, w_ref[...], 0.0)
    sumsq = jnp.sum(w * w, axis=0, keepdims=True)
    scale = SCALE * jax.lax.rsqrt(jnp.maximum(sumsq, 1e-30))
    ws = (w * scale).astype(jnp.bfloat16)

    dots = jax.lax.dot_general(
        x_ref[...], ws,
        (((1,), (0,)), ((), ())),
        preferred_element_type=jnp.float32)

    acc_se_ref[...] += jnp.exp(dots)
    lmask = lab_ref[...] == cid
    acc_ll_ref[...] += jnp.where(lmask, dots, 0.0)

    @pl.when(i == grid_n - 1)
    def _finalize():
        b = x_ref.shape[0] // 2
        n_pad = grid_n * tile_c - num_classes
        se = jnp.sum(acc_se_ref[...], axis=1, keepdims=True) - n_pad
        ll = jnp.sum(acc_ll_ref[...], axis=1, keepdims=True)
        ce = jnp.log(se) - ll
        v_loss = jnp.sum(ce[:b]) / b
        t_loss = jnp.sum(ce[b:]) / b

        sim = jax.lax.dot_general(
            x_ref[0:b, :], tt_ref[...],
            (((1,), (0,)), ((), ())),
            preferred_element_type=jnp.float32)
        lmat = lab_ref[0:b, :] == labrow_ref[0:1, :]
        loss_pos = jnp.log1p(jnp.exp(-SCALE_POS * (sim - ALPHA)))
        loss_neg = jnp.log1p(jnp.exp(SCALE_NEG * (sim - BETA)))
        ga = 2.0 * jnp.sum(jnp.where(lmat, loss_pos, loss_neg)) / b

        col = jax.lax.broadcasted_iota(jnp.int32, (8, 128), 1)
        res = jnp.where(col == 0, v_loss + t_loss,
              jnp.where(col == 1, ga,
              jnp.where(col == 2, v_loss, t_loss)))
        out_ref[...] = res


def kernel(visual_embed, textual_embed, labels, W):
    batch, feat = visual_embed.shape
    num_classes = W.shape[1]
    tile_c = 1024
    grid_n = (num_classes + tile_c - 1) // tile_c

    x = jnp.concatenate([visual_embed, textual_embed], axis=0)
    x = x.astype(jnp.bfloat16)
    tt = textual_embed.T.astype(jnp.bfloat16)
    lab = jnp.concatenate([labels, labels]).astype(jnp.int32)
    lab = lab.reshape(2 * batch, 1)
    labrow = jnp.broadcast_to(labels.astype(jnp.int32)[None, :], (8, batch))

    out = pl.pallas_call(
        functools.partial(_loss_kernel, num_classes, grid_n, tile_c),
        grid=(grid_n,),
        in_specs=[
            pl.BlockSpec((2 * batch, feat), lambda i: (0, 0)),
            pl.BlockSpec((feat, tile_c), lambda i: (0, i)),
            pl.BlockSpec((2 * batch, 1), lambda i: (0, 0)),
            pl.BlockSpec((8, batch), lambda i: (0, 0)),
            pl.BlockSpec((feat, batch), lambda i: (0, 0)),
        ],
        out_specs=pl.BlockSpec((8, 128), lambda i: (0, 0)),
        out_shape=jax.ShapeDtypeStruct((8, 128), jnp.float32),
        scratch_shapes=[
            pltpu.VMEM((2 * batch, tile_c), jnp.float32),
            pltpu.VMEM((2 * batch, tile_c), jnp.float32),
        ],
    )(x, W, lab, labrow, tt)

    instance_loss = out[0, 0]
    global_align_loss = out[0, 1]
    v_loss = out[0, 2]
    t_loss = out[0, 3]
    return (instance_loss, global_align_loss, v_loss, t_loss)
